# baseline (speedup 1.0000x reference)
import functools

import jax
import jax.numpy as jnp
from jax import lax
from jax.experimental import pallas as pl
from jax.experimental.pallas import tpu as pltpu
from jax.experimental.pallas import tpu_sc as plsc

_B = 16384
_D = 64
_NC = 2
_NS = 16
_NW = _NC * _NS
_BW = _B // _NW
_G = _BW // 16


def _dist_body(src_hbm, dst_hbm, table_hbm, out_hbm,
               idx_s, idx_d, rows_s, rows_d, part_v, out_v, sem_s, sem_d):
    wid = lax.axis_index("s") * _NC + lax.axis_index("c")
    base = wid * _BW

    pltpu.sync_copy(src_hbm.at[pl.ds(base, _BW)], idx_s)
    pltpu.sync_copy(dst_hbm.at[pl.ds(base, _BW)], idx_d)
    cp_s = pltpu.async_copy(table_hbm.at[idx_s], rows_s, sem_s)
    cp_d = pltpu.async_copy(table_hbm.at[idx_d], rows_d, sem_d)
    cp_s.wait()
    cp_d.wait()

    lane16 = lax.iota(jnp.int32, 16) * 16

    def group(g, carry):
        for r in range(16):
            row = g * 16 + r
            p = jnp.zeros((16,), jnp.float32)
            for c in range(0, _D, 16):
                s = rows_s[row, pl.ds(c, 16)]
                t = rows_d[row, pl.ds(c, 16)]
                df = s - t
                p = p + df * df
            part_v[pl.ds(r * 16, 16)] = p
        acc = jnp.zeros((16,), jnp.float32)
        for k in range(16):
            acc = acc + plsc.load_gather(part_v, [lane16 + k])
        acc = acc + 1e-12
        yi = 0x5F3759DF - (plsc.bitcast(acc, jnp.int32) >> 1)
        y = plsc.bitcast(yi, jnp.float32)
        y = y * (1.5 - 0.5 * acc * y * y)
        y = y * (1.5 - 0.5 * acc * y * y)
        y = y * (1.5 - 0.5 * acc * y * y)
        out_v[pl.ds(g * 16, 16)] = acc * y
        return carry

    lax.fori_loop(0, _G, group, 0)
    pltpu.sync_copy(out_v, out_hbm.at[pl.ds(base, _BW)])


_dist_kernel = functools.partial(
    pl.kernel,
    mesh=plsc.VectorSubcoreMesh(core_axis_name="c", subcore_axis_name="s"),
    out_type=jax.ShapeDtypeStruct((_B,), jnp.float32),
    compiler_params=pltpu.CompilerParams(
        needs_layout_passes=False, use_tc_tiling_on_sc=False),
    scratch_types=[
        pltpu.VMEM((_BW,), jnp.int32),
        pltpu.VMEM((_BW,), jnp.int32),
        pltpu.VMEM((_BW, _D), jnp.float32),
        pltpu.VMEM((_BW, _D), jnp.float32),
        pltpu.VMEM((256,), jnp.float32),
        pltpu.VMEM((_BW,), jnp.float32),
        pltpu.SemaphoreType.DMA,
        pltpu.SemaphoreType.DMA,
    ],
)(_dist_body)


def kernel(input_triplet, embeds):
    src = input_triplet[:, 0]
    dst = input_triplet[:, 1]
    return _dist_kernel(src, dst, embeds)


# speedup vs baseline: 2.5043x; 2.5043x over previous
import functools

import jax
import jax.numpy as jnp
from jax import lax
from jax.experimental import pallas as pl
from jax.experimental.pallas import tpu as pltpu
from jax.experimental.pallas import tpu_sc as plsc

_B = 16384
_D = 64
_NC = 2
_NS = 16
_NW = _NC * _NS
_BW = _B // _NW
_K = 256
_NCHUNK = _BW // _K


def _dist_body(src_hbm, dst_hbm, table_hbm, out_hbm,
               idx_s_v, idx_d_v, rows_s, rows_d, part_v, out_v, sem):
    wid = lax.axis_index("s") * _NC + lax.axis_index("c")
    base = wid * _BW

    pltpu.sync_copy(src_hbm.at[pl.ds(base, _BW)], idx_s_v)
    pltpu.sync_copy(dst_hbm.at[pl.ds(base, _BW)], idx_d_v)

    lane16 = lax.iota(jnp.int32, 16) * 16

    for chunk in range(_NCHUNK):
        off = chunk * _K

        def issue(i16, _):
            vs = idx_s_v[pl.ds(off + i16 * 16, 16)] >> 3
            ws = idx_s_v[pl.ds(off + i16 * 16, 16)] & 7
            vd = idx_d_v[pl.ds(off + i16 * 16, 16)] >> 3
            wd = idx_d_v[pl.ds(off + i16 * 16, 16)] & 7
            for j in range(16):
                pltpu.async_copy(table_hbm.at[vs[j], ws[j]],
                                 rows_s.at[i16 * 16 + j], sem)
                pltpu.async_copy(table_hbm.at[vd[j], wd[j]],
                                 rows_d.at[i16 * 16 + j], sem)
            return 0

        lax.fori_loop(0, _K // 16, issue, 0)

        def drain(i, _):
            pltpu.make_async_copy(table_hbm.at[0, 0], rows_s.at[0], sem).wait()
            pltpu.make_async_copy(table_hbm.at[0, 0], rows_d.at[0], sem).wait()
            return 0

        lax.fori_loop(0, _K, drain, 0)

        def group(g, _):
            for rloc in range(16):
                row = g * 16 + rloc
                p = jnp.zeros((16,), jnp.float32)
                for c in range(0, _D, 16):
                    s = rows_s[row, pl.ds(c, 16)]
                    t = rows_d[row, pl.ds(c, 16)]
                    df = s - t
                    p = p + df * df
                part_v[pl.ds(rloc * 16, 16)] = p
            acc = jnp.zeros((16,), jnp.float32)
            for k in range(16):
                acc = acc + plsc.load_gather(part_v, [lane16 + k])
            acc = acc + 1e-12
            yi = 0x5F3759DF - (plsc.bitcast(acc, jnp.int32) >> 1)
            y = plsc.bitcast(yi, jnp.float32)
            y = y * (1.5 - 0.5 * acc * y * y)
            y = y * (1.5 - 0.5 * acc * y * y)
            y = y * (1.5 - 0.5 * acc * y * y)
            out_v[pl.ds(off + g * 16, 16)] = acc * y
            return 0

        lax.fori_loop(0, _K // 16, group, 0)

    pltpu.sync_copy(out_v, out_hbm.at[pl.ds(base, _BW)])


_dist_kernel = functools.partial(
    pl.kernel,
    mesh=plsc.VectorSubcoreMesh(core_axis_name="c", subcore_axis_name="s"),
    out_type=jax.ShapeDtypeStruct((_B,), jnp.float32),
    compiler_params=pltpu.CompilerParams(needs_layout_passes=False),
    scratch_types=[
        pltpu.VMEM((_BW,), jnp.int32),
        pltpu.VMEM((_BW,), jnp.int32),
        pltpu.VMEM((_K, _D), jnp.float32),
        pltpu.VMEM((_K, _D), jnp.float32),
        pltpu.VMEM((256,), jnp.float32),
        pltpu.VMEM((_BW,), jnp.float32),
        pltpu.SemaphoreType.DMA,
    ],
)(_dist_body)


def kernel(input_triplet, embeds):
    src = input_triplet[:, 0]
    dst = input_triplet[:, 1]
    table3 = embeds.reshape(125000, 8, _D)
    return _dist_kernel(src, dst, table3)
